# baseline (speedup 1.0000x reference)
import functools

import jax
import jax.numpy as jnp
from jax import lax
from jax.experimental import pallas as pl
from jax.experimental.pallas import tpu as pltpu
from jax.experimental.pallas import tpu_sc as plsc

_N = 10000
_E = 320000
_D = 128
_H = _N // 2
_NC = 2
_NS = 16
_NW = _NC * _NS
_EPW = 10112
_EP = _NW * _EPW
_CAP = 2944
_CAPB = _CAP + 32
_CH = 64
_NIT = 2 * _CAP // _CH
_HP = _H + 8
_RPT = 312
_RLAST = _H - (_NS - 1) * _RPT

_mesh = plsc.VectorSubcoreMesh(core_axis_name="c", subcore_axis_name="s")


def _sc_bucket(src, dst):
    @functools.partial(
        pl.kernel,
        out_type=(
            jax.ShapeDtypeStruct((4 * _NW * _CAP,), jnp.int32),
            jax.ShapeDtypeStruct((4 * _NW * _CAP,), jnp.int32),
        ),
        mesh=_mesh,
        scratch_types=[
            pltpu.VMEM((_EPW,), jnp.int32),
            pltpu.VMEM((_EPW,), jnp.int32),
            pltpu.VMEM((_EPW,), jnp.int32),
            pltpu.VMEM((_CAP,), jnp.int32),
            pltpu.VMEM((_CAP,), jnp.int32),
            pltpu.VMEM((48,), jnp.int32),
            pltpu.VMEM((32,), jnp.int32),
            pltpu.VMEM((128,), jnp.int32),
            pltpu.VMEM((128,), jnp.int32),
            pltpu.SemaphoreType.DMA((2,)),
        ],
    )
    def bucket_kernel(src_hbm, dst_hbm, sl_hbm, dl_hbm, sfull, dfull,
                      posbuf, junk_sbuf, junk_dbuf, lad, rot,
                      posw0, posw1, ssem):
        cid = lax.axis_index("c")
        sid = lax.axis_index("s")
        wid = cid * _NS + sid

        ebase = pl.multiple_of(wid * _EPW, 8)
        pltpu.sync_copy(src_hbm.at[pl.ds(ebase, _EPW)], sfull)
        pltpu.sync_copy(dst_hbm.at[pl.ds(ebase, _EPW)], dfull)

        iota = lax.iota(jnp.int32, 16)
        junk_s = jnp.full((16,), _H, jnp.int32)
        junk_d = _H + lax.rem(iota, 8)

        def prefill(k, carry):
            junk_sbuf[pl.ds(k * 16, 16)] = junk_s
            junk_dbuf[pl.ds(k * 16, 16)] = junk_d
            return carry

        lax.fori_loop(0, _CAP // 16, prefill, 0)
        lad[pl.ds(0, 16)] = jnp.zeros((16,), jnp.int32)

        for b in range(4):
            off = pl.multiple_of((b * _NW + wid) * _CAP, 8)
            pltpu.sync_copy(junk_sbuf, sl_hbm.at[pl.ds(off, _CAP)])
            pltpu.sync_copy(junk_dbuf, dl_hbm.at[pl.ds(off, _CAP)])

        bases = [jnp.full((16,), 0, jnp.int32) + (b * _NW + wid) * _CAP
                 for b in range(4)]

        def scan(i, counts):
            sv = sfull[pl.ds(i * 16, 16)]
            dv = dfull[pl.ds(i * 16, 16)]
            shi = 1 + ((sv - _H) >> 31)
            dhi = 1 + ((dv - _H) >> 31)
            sfull[pl.ds(i * 16, 16)] = sv - shi * _H
            dfull[pl.ds(i * 16, 16)] = dv - dhi * _H
            slo = 1 - shi
            dlo = 1 - dhi
            m = (slo * dlo, slo * dhi, shi * dlo, shi * dhi)
            mpack = m[0] + (m[1] << 8) + (m[2] << 16) + (m[3] << 24)

            x = mpack
            for k in (1, 2, 4, 8):
                lad[pl.ds(16, 16)] = x
                x = x + lad[pl.ds(16 - k, 16)]
            excl = x - mpack

            y = mpack
            for k in (8, 4, 2, 1):
                rot[pl.ds(0, 16)] = y
                rot[pl.ds(16, 16)] = y
                y = y + rot[pl.ds(k, 16)]

            pos = jnp.zeros((16,), jnp.int32)
            new_counts = []
            for b in range(4):
                e_b = (excl >> (8 * b)) & 255
                pos = pos + m[b] * (counts[b] + e_b)
                new_counts.append(counts[b] + ((y >> (8 * b)) & 255))
            posbuf[pl.ds(i * 16, 16)] = pos
            return tuple(new_counts)

        lax.fori_loop(0, _EPW // 16, scan, tuple(bases))

        nsc = _EPW // 128

        def stage(k, posw):
            for v in range(8):
                posw[pl.ds(v * 16, 16)] = posbuf[pl.ds(k * 128 + v * 16, 16)]

        def fire(k, posw, sem):
            eoff = pl.multiple_of(k * 128, 8)
            pltpu.async_copy(sfull.at[pl.ds(eoff, 128)],
                             sl_hbm.at[posw], sem)
            pltpu.async_copy(dfull.at[pl.ds(eoff, 128)],
                             dl_hbm.at[posw], sem)

        def drain(k, posw, sem):
            eoff = pl.multiple_of(k * 128, 8)
            pltpu.make_async_copy(sfull.at[pl.ds(eoff, 128)],
                                  sl_hbm.at[posw], sem).wait()
            pltpu.make_async_copy(dfull.at[pl.ds(eoff, 128)],
                                  dl_hbm.at[posw], sem).wait()

        stage(0, posw0)
        fire(0, posw0, ssem.at[0])

        def sbody(k, carry):
            @pl.when(lax.rem(k, 2) == 0)
            def _():
                @pl.when(k + 1 < nsc)
                def _():
                    stage(k + 1, posw1)
                    fire(k + 1, posw1, ssem.at[1])
                drain(k, posw0, ssem.at[0])

            @pl.when(lax.rem(k, 2) == 1)
            def _():
                @pl.when(k + 1 < nsc)
                def _():
                    stage(k + 1, posw0)
                    fire(k + 1, posw0, ssem.at[0])
                drain(k, posw1, ssem.at[1])

            return carry

        lax.fori_loop(0, nsc, sbody, 0)

    return bucket_kernel(src, dst)


def _sc_agg(h, slist, dlist):
    @functools.partial(
        pl.kernel,
        out_type=jax.ShapeDtypeStruct((_N, _D), jnp.float32),
        mesh=_mesh,
        scratch_types=[
            pltpu.VMEM_SHARED((_HP, _D), jnp.float32),
            pltpu.VMEM_SHARED((_HP, _D), jnp.float32),
            pltpu.VMEM((_CH,), jnp.int32),
            pltpu.VMEM((_CH,), jnp.int32),
            pltpu.VMEM((_CH,), jnp.int32),
            pltpu.VMEM((_CH,), jnp.int32),
            pltpu.VMEM((_CH,), jnp.int32),
            pltpu.VMEM((_CH,), jnp.int32),
            pltpu.VMEM((_CH, _D), jnp.float32),
            pltpu.VMEM((_CH, _D), jnp.float32),
            pltpu.VMEM((_CH, _D), jnp.float32),
            pltpu.SemaphoreType.DMA((3,)),
            pltpu.SemaphoreType.DMA((3,)),
        ],
    )
    def agg_kernel(h_hbm, sl_hbm, dl_hbm, out_hbm,
                   acc, table, s_0, s_1, s_2, d_0, d_1, d_2,
                   rows_0, rows_1, rows_2, gsem, isem):
        cid = lax.axis_index("c")
        sid = lax.axis_index("s")
        row0 = pl.multiple_of(sid * _RPT, 8)
        sets = ((s_0, d_0, rows_0), (s_1, d_1, rows_1), (s_2, d_2, rows_2))

        def half_copy(src_ref, src_base, dst_ref, dst_base):
            @pl.when(sid < _NS - 1)
            def _():
                pltpu.sync_copy(
                    src_ref.at[pl.ds(pl.multiple_of(src_base + row0, 8),
                                     _RPT)],
                    dst_ref.at[pl.ds(pl.multiple_of(dst_base + row0, 8),
                                     _RPT)])

            @pl.when(sid == _NS - 1)
            def _():
                last = (_NS - 1) * _RPT
                pltpu.sync_copy(
                    src_ref.at[pl.ds(pl.multiple_of(src_base + last, 8),
                                     _RLAST)],
                    dst_ref.at[pl.ds(pl.multiple_of(dst_base + last, 8),
                                     _RLAST)])

        half_copy(h_hbm, cid * _H, acc, 0)

        for p in (0, 1):
            base0 = pl.multiple_of(((2 * p + cid) * _NW + 2 * sid) * _CAP, 8)

            def idx_off(i):
                return pl.multiple_of(base0 + i * _CH, 8)

            def issue_idx(i, s_buf, d_buf, i_sem):
                off = idx_off(i)
                pltpu.async_copy(sl_hbm.at[pl.ds(off, _CH)], s_buf, i_sem)
                pltpu.async_copy(dl_hbm.at[pl.ds(off, _CH)], d_buf, i_sem)

            def wait_idx(i, s_buf, d_buf, i_sem):
                off = idx_off(i)
                pltpu.make_async_copy(sl_hbm.at[pl.ds(off, _CH)], s_buf,
                                      i_sem).wait()
                pltpu.make_async_copy(dl_hbm.at[pl.ds(off, _CH)], d_buf,
                                      i_sem).wait()

            plsc.subcore_barrier()
            half_copy(h_hbm, p * _H, table, 0)
            issue_idx(0, s_0, d_0, isem.at[0])
            issue_idx(1, s_1, d_1, isem.at[1])
            issue_idx(2, s_2, d_2, isem.at[2])
            plsc.subcore_barrier()

            wait_idx(0, s_0, d_0, isem.at[0])
            pltpu.async_copy(table.at[s_0], rows_0, gsem.at[0])
            wait_idx(1, s_1, d_1, isem.at[1])
            pltpu.async_copy(table.at[s_1], rows_1, gsem.at[1])

            def step(i, c, n, p_):
                s_c, d_c, rows_c = sets[c]
                s_p, d_p, rows_p = sets[p_]
                pltpu.make_async_copy(table.at[s_c], rows_c,
                                      gsem.at[c]).wait()

                @pl.when(i + 2 < _NIT)
                def _():
                    wait_idx(i + 2, s_p, d_p, isem.at[p_])
                    pltpu.async_copy(table.at[s_p], rows_p, gsem.at[p_])

                pltpu.sync_copy(rows_c, acc.at[d_c], add=True)

                @pl.when(i + 3 < _NIT)
                def _():
                    issue_idx(i + 3, s_c, d_c, isem.at[c])

            def body(i, carry):
                @pl.when(lax.rem(i, 3) == 0)
                def _():
                    step(i, 0, 1, 2)

                @pl.when(lax.rem(i, 3) == 1)
                def _():
                    step(i, 1, 2, 0)

                @pl.when(lax.rem(i, 3) == 2)
                def _():
                    step(i, 2, 0, 1)

                return carry

            lax.fori_loop(0, _NIT, body, 0)

        plsc.subcore_barrier()
        half_copy(acc, 0, out_hbm, cid * _H)

    return agg_kernel(h, slist, dlist)


_BLK = 1000


def _tc_layer(z, Wa, ba, Wb, bb):
    def body(z_ref, wa_ref, ba_ref, wb_ref, bb_ref, out_ref):
        t = jnp.dot(z_ref[...], wa_ref[...], preferred_element_type=jnp.float32)
        t = jnp.maximum(t + ba_ref[...], 0.0)
        t = jnp.dot(t, wb_ref[...], preferred_element_type=jnp.float32)
        out_ref[...] = jnp.maximum(t + bb_ref[...], 0.0)

    return pl.pallas_call(
        body,
        grid=(_N // _BLK,),
        in_specs=[
            pl.BlockSpec((_BLK, _D), lambda i: (i, 0)),
            pl.BlockSpec((_D, _D), lambda i: (0, 0)),
            pl.BlockSpec((1, _D), lambda i: (0, 0)),
            pl.BlockSpec((_D, _D), lambda i: (0, 0)),
            pl.BlockSpec((1, _D), lambda i: (0, 0)),
        ],
        out_specs=pl.BlockSpec((_BLK, _D), lambda i: (i, 0)),
        out_shape=jax.ShapeDtypeStruct((_N, _D), jnp.float32),
    )(z, Wa, ba.reshape(1, _D), Wb, bb.reshape(1, _D))


def kernel(x, edges, W1, b1, W2, b2, W3, b3, W4, b4, W5, b5, W6, b6,
           W7, b7, W8, b8):
    epw_real = _E // _NW
    padb = jnp.full((_NW, _EPW - epw_real), _N, jnp.int32)
    src = jnp.concatenate([edges[0].reshape(_NW, epw_real), padb], axis=1)
    dst = jnp.concatenate([edges[1].reshape(_NW, epw_real), padb], axis=1)
    slist, dlist = _sc_bucket(src.reshape(-1), dst.reshape(-1))
    h = x
    for Wa, ba, Wb, bb in ((W1, b1, W2, b2), (W3, b3, W4, b4),
                           (W5, b5, W6, b6), (W7, b7, W8, b8)):
        z = _sc_agg(h, slist, dlist)
        h = _tc_layer(z, Wa, ba, Wb, bb)
    return h


# speedup vs baseline: 2.0700x; 2.0700x over previous
import functools

import jax
import jax.numpy as jnp
from jax import lax
from jax.experimental import pallas as pl
from jax.experimental.pallas import tpu as pltpu
from jax.experimental.pallas import tpu_sc as plsc

_N = 10000
_E = 320000
_D = 128
_H = _N // 2
_NC = 2
_NS = 16
_NW = _NC * _NS
_EPW = 10112
_EP = _NW * _EPW
_CAP = 2944
_CAPB = _CAP + 32
_CH = 64
_NIT = 2 * _CAP // _CH
_HP = _H + 8
_RPT = 312
_RLAST = _H - (_NS - 1) * _RPT

_mesh = plsc.VectorSubcoreMesh(core_axis_name="c", subcore_axis_name="s")


def _sc_bucket(src, dst):
    @functools.partial(
        pl.kernel,
        out_type=(
            jax.ShapeDtypeStruct((4 * _NW * _CAP,), jnp.int32),
            jax.ShapeDtypeStruct((4 * _NW * _CAP,), jnp.int32),
        ),
        mesh=_mesh,
        scratch_types=[
            pltpu.VMEM((_EPW,), jnp.int32),
            pltpu.VMEM((_EPW,), jnp.int32),
            pltpu.VMEM((_EPW,), jnp.int32),
            pltpu.VMEM((_CAP,), jnp.int32),
            pltpu.VMEM((_CAP,), jnp.int32),
            pltpu.VMEM((48,), jnp.int32),
            pltpu.VMEM((32,), jnp.int32),
            pltpu.VMEM((128,), jnp.int32),
            pltpu.VMEM((128,), jnp.int32),
            pltpu.VMEM_SHARED((_NS * 4 * _CAPB,), jnp.int32),
            pltpu.VMEM_SHARED((_NS * 4 * _CAPB,), jnp.int32),
            pltpu.SemaphoreType.DMA((2,)),
        ],
    )
    def bucket_kernel(src_hbm, dst_hbm, sl_hbm, dl_hbm, sfull, dfull,
                      posbuf, junk_sbuf, junk_dbuf, lad, rot,
                      posw0, posw1, sstage, dstage, ssem):
        cid = lax.axis_index("c")
        sid = lax.axis_index("s")
        wid = cid * _NS + sid

        ebase = pl.multiple_of(wid * _EPW, 8)
        pltpu.sync_copy(src_hbm.at[pl.ds(ebase, _EPW)], sfull)
        pltpu.sync_copy(dst_hbm.at[pl.ds(ebase, _EPW)], dfull)

        iota = lax.iota(jnp.int32, 16)
        junk_s = jnp.full((16,), _H, jnp.int32)
        junk_d = _H + lax.rem(iota, 8)

        def prefill(k, carry):
            junk_sbuf[pl.ds(k * 16, 16)] = junk_s
            junk_dbuf[pl.ds(k * 16, 16)] = junk_d
            return carry

        lax.fori_loop(0, _CAP // 16, prefill, 0)
        lad[pl.ds(0, 16)] = jnp.zeros((16,), jnp.int32)

        for b in range(4):
            off = pl.multiple_of((sid * 4 + b) * _CAPB, 8)
            pltpu.sync_copy(junk_sbuf, sstage.at[pl.ds(off, _CAP)])
            pltpu.sync_copy(junk_dbuf, dstage.at[pl.ds(off, _CAP)])

        bases = [jnp.full((16,), 0, jnp.int32) + (sid * 4 + b) * _CAPB
                 for b in range(4)]

        def scan(i, counts):
            sv = sfull[pl.ds(i * 16, 16)]
            dv = dfull[pl.ds(i * 16, 16)]
            shi = 1 + ((sv - _H) >> 31)
            dhi = 1 + ((dv - _H) >> 31)
            sfull[pl.ds(i * 16, 16)] = sv - shi * _H
            dfull[pl.ds(i * 16, 16)] = dv - dhi * _H
            slo = 1 - shi
            dlo = 1 - dhi
            m = (slo * dlo, slo * dhi, shi * dlo, shi * dhi)
            mpack = m[0] + (m[1] << 8) + (m[2] << 16) + (m[3] << 24)

            lad[pl.ds(16, 16)] = mpack
            excl = jnp.zeros((16,), jnp.int32)
            for k in range(1, 16):
                excl = excl + lad[pl.ds(16 - k, 16)]

            rot[pl.ds(0, 16)] = mpack
            rot[pl.ds(16, 16)] = mpack
            y = mpack
            for k in range(1, 16):
                y = y + rot[pl.ds(k, 16)]

            pos = jnp.zeros((16,), jnp.int32)
            new_counts = []
            for b in range(4):
                e_b = (excl >> (8 * b)) & 255
                pos = pos + m[b] * (counts[b] + e_b)
                new_counts.append(counts[b] + ((y >> (8 * b)) & 255))
            posbuf[pl.ds(i * 16, 16)] = pos
            return tuple(new_counts)

        lax.fori_loop(0, _EPW // 16, scan, tuple(bases))

        nsc = _EPW // 128

        def stage(k, posw):
            for v in range(8):
                posw[pl.ds(v * 16, 16)] = posbuf[pl.ds(k * 128 + v * 16, 16)]

        def fire(k, posw, sem):
            eoff = pl.multiple_of(k * 128, 8)
            pltpu.async_copy(sfull.at[pl.ds(eoff, 128)],
                             sstage.at[posw], sem)
            pltpu.async_copy(dfull.at[pl.ds(eoff, 128)],
                             dstage.at[posw], sem)

        def drain(k, posw, sem):
            eoff = pl.multiple_of(k * 128, 8)
            pltpu.make_async_copy(sfull.at[pl.ds(eoff, 128)],
                                  sstage.at[posw], sem).wait()
            pltpu.make_async_copy(dfull.at[pl.ds(eoff, 128)],
                                  dstage.at[posw], sem).wait()

        stage(0, posw0)
        fire(0, posw0, ssem.at[0])

        def sbody(k, carry):
            @pl.when(lax.rem(k, 2) == 0)
            def _():
                @pl.when(k + 1 < nsc)
                def _():
                    stage(k + 1, posw1)
                    fire(k + 1, posw1, ssem.at[1])
                drain(k, posw0, ssem.at[0])

            @pl.when(lax.rem(k, 2) == 1)
            def _():
                @pl.when(k + 1 < nsc)
                def _():
                    stage(k + 1, posw0)
                    fire(k + 1, posw0, ssem.at[0])
                drain(k, posw1, ssem.at[1])

            return carry

        lax.fori_loop(0, nsc, sbody, 0)

        for b in range(4):
            soff = pl.multiple_of((sid * 4 + b) * _CAPB, 8)
            hoff = pl.multiple_of((b * _NW + wid) * _CAP, 8)
            pltpu.sync_copy(sstage.at[pl.ds(soff, _CAP)], junk_sbuf)
            pltpu.sync_copy(junk_sbuf, sl_hbm.at[pl.ds(hoff, _CAP)])
            pltpu.sync_copy(dstage.at[pl.ds(soff, _CAP)], junk_dbuf)
            pltpu.sync_copy(junk_dbuf, dl_hbm.at[pl.ds(hoff, _CAP)])

    return bucket_kernel(src, dst)


def _sc_agg(h, slist, dlist):
    @functools.partial(
        pl.kernel,
        out_type=jax.ShapeDtypeStruct((_N, _D), jnp.float32),
        mesh=_mesh,
        scratch_types=[
            pltpu.VMEM_SHARED((_HP, _D), jnp.float32),
            pltpu.VMEM_SHARED((_HP, _D), jnp.float32),
            pltpu.VMEM((_CH,), jnp.int32),
            pltpu.VMEM((_CH,), jnp.int32),
            pltpu.VMEM((_CH,), jnp.int32),
            pltpu.VMEM((_CH,), jnp.int32),
            pltpu.VMEM((_CH,), jnp.int32),
            pltpu.VMEM((_CH,), jnp.int32),
            pltpu.VMEM((_CH, _D), jnp.float32),
            pltpu.VMEM((_CH, _D), jnp.float32),
            pltpu.VMEM((_CH, _D), jnp.float32),
            pltpu.SemaphoreType.DMA((3,)),
            pltpu.SemaphoreType.DMA((3,)),
        ],
    )
    def agg_kernel(h_hbm, sl_hbm, dl_hbm, out_hbm,
                   acc, table, s_0, s_1, s_2, d_0, d_1, d_2,
                   rows_0, rows_1, rows_2, gsem, isem):
        cid = lax.axis_index("c")
        sid = lax.axis_index("s")
        row0 = pl.multiple_of(sid * _RPT, 8)
        sets = ((s_0, d_0, rows_0), (s_1, d_1, rows_1), (s_2, d_2, rows_2))

        def half_copy(src_ref, src_base, dst_ref, dst_base):
            @pl.when(sid < _NS - 1)
            def _():
                pltpu.sync_copy(
                    src_ref.at[pl.ds(pl.multiple_of(src_base + row0, 8),
                                     _RPT)],
                    dst_ref.at[pl.ds(pl.multiple_of(dst_base + row0, 8),
                                     _RPT)])

            @pl.when(sid == _NS - 1)
            def _():
                last = (_NS - 1) * _RPT
                pltpu.sync_copy(
                    src_ref.at[pl.ds(pl.multiple_of(src_base + last, 8),
                                     _RLAST)],
                    dst_ref.at[pl.ds(pl.multiple_of(dst_base + last, 8),
                                     _RLAST)])

        half_copy(h_hbm, cid * _H, acc, 0)

        for p in (0, 1):
            base0 = pl.multiple_of(((2 * p + cid) * _NW + 2 * sid) * _CAP, 8)

            def idx_off(i):
                return pl.multiple_of(base0 + i * _CH, 8)

            def issue_idx(i, s_buf, d_buf, i_sem):
                off = idx_off(i)
                pltpu.async_copy(sl_hbm.at[pl.ds(off, _CH)], s_buf, i_sem)
                pltpu.async_copy(dl_hbm.at[pl.ds(off, _CH)], d_buf, i_sem)

            def wait_idx(i, s_buf, d_buf, i_sem):
                off = idx_off(i)
                pltpu.make_async_copy(sl_hbm.at[pl.ds(off, _CH)], s_buf,
                                      i_sem).wait()
                pltpu.make_async_copy(dl_hbm.at[pl.ds(off, _CH)], d_buf,
                                      i_sem).wait()

            plsc.subcore_barrier()
            half_copy(h_hbm, p * _H, table, 0)
            issue_idx(0, s_0, d_0, isem.at[0])
            issue_idx(1, s_1, d_1, isem.at[1])
            issue_idx(2, s_2, d_2, isem.at[2])
            plsc.subcore_barrier()

            wait_idx(0, s_0, d_0, isem.at[0])
            pltpu.async_copy(table.at[s_0], rows_0, gsem.at[0])
            wait_idx(1, s_1, d_1, isem.at[1])
            pltpu.async_copy(table.at[s_1], rows_1, gsem.at[1])

            def step(i, c, n, p_):
                s_c, d_c, rows_c = sets[c]
                s_p, d_p, rows_p = sets[p_]
                pltpu.make_async_copy(table.at[s_c], rows_c,
                                      gsem.at[c]).wait()

                @pl.when(i + 2 < _NIT)
                def _():
                    wait_idx(i + 2, s_p, d_p, isem.at[p_])
                    pltpu.async_copy(table.at[s_p], rows_p, gsem.at[p_])

                pltpu.sync_copy(rows_c, acc.at[d_c], add=True)

                @pl.when(i + 3 < _NIT)
                def _():
                    issue_idx(i + 3, s_c, d_c, isem.at[c])

            def body(i, carry):
                @pl.when(lax.rem(i, 3) == 0)
                def _():
                    step(i, 0, 1, 2)

                @pl.when(lax.rem(i, 3) == 1)
                def _():
                    step(i, 1, 2, 0)

                @pl.when(lax.rem(i, 3) == 2)
                def _():
                    step(i, 2, 0, 1)

                return carry

            lax.fori_loop(0, _NIT, body, 0)

        plsc.subcore_barrier()
        half_copy(acc, 0, out_hbm, cid * _H)

    return agg_kernel(h, slist, dlist)


_BLK = 1000


def _tc_layer(z, Wa, ba, Wb, bb):
    def body(z_ref, wa_ref, ba_ref, wb_ref, bb_ref, out_ref):
        t = jnp.dot(z_ref[...], wa_ref[...], preferred_element_type=jnp.float32)
        t = jnp.maximum(t + ba_ref[...], 0.0)
        t = jnp.dot(t, wb_ref[...], preferred_element_type=jnp.float32)
        out_ref[...] = jnp.maximum(t + bb_ref[...], 0.0)

    return pl.pallas_call(
        body,
        grid=(_N // _BLK,),
        in_specs=[
            pl.BlockSpec((_BLK, _D), lambda i: (i, 0)),
            pl.BlockSpec((_D, _D), lambda i: (0, 0)),
            pl.BlockSpec((1, _D), lambda i: (0, 0)),
            pl.BlockSpec((_D, _D), lambda i: (0, 0)),
            pl.BlockSpec((1, _D), lambda i: (0, 0)),
        ],
        out_specs=pl.BlockSpec((_BLK, _D), lambda i: (i, 0)),
        out_shape=jax.ShapeDtypeStruct((_N, _D), jnp.float32),
    )(z, Wa, ba.reshape(1, _D), Wb, bb.reshape(1, _D))


def kernel(x, edges, W1, b1, W2, b2, W3, b3, W4, b4, W5, b5, W6, b6,
           W7, b7, W8, b8):
    epw_real = _E // _NW
    padb = jnp.full((_NW, _EPW - epw_real), _N, jnp.int32)
    src = jnp.concatenate([edges[0].reshape(_NW, epw_real), padb], axis=1)
    dst = jnp.concatenate([edges[1].reshape(_NW, epw_real), padb], axis=1)
    slist, dlist = _sc_bucket(src.reshape(-1), dst.reshape(-1))
    h = x
    for Wa, ba, Wb, bb in ((W1, b1, W2, b2), (W3, b3, W4, b4),
                           (W5, b5, W6, b6), (W7, b7, W8, b8)):
        z = _sc_agg(h, slist, dlist)
        h = _tc_layer(z, Wa, ba, Wb, bb)
    return h
